# baseline (speedup 1.0000x reference)
import functools

import jax
import jax.numpy as jnp
import numpy as np
from jax import lax
from jax.experimental import pallas as pl
from jax.experimental.pallas import tpu as pltpu
from jax.experimental.pallas import tpu_sc as plsc

_N = 10000
_E = 320000
_D = 128
_H = 8
_DH = 16

_CHUNK = 64
_NCHUNK = _E // _CHUNK
_NW = 32
_NSUB = 16
_NPAD = 10112
_WV_PER_SUB = _NPAD // _NSUB
_NZ = 1280
_Z_PER_SUB = _NZ // _NSUB


def _pre_body(x_ref, wq_ref, wk_ref, wv_ref, we_ref, q_ref, kw_ref, v_ref):
    x = x_ref[...]
    q_ref[...] = jnp.dot(x, wq_ref[...], preferred_element_type=jnp.float32)
    k = jnp.dot(x, wk_ref[...], preferred_element_type=jnp.float32)
    v_ref[...] = jnp.dot(x, wv_ref[...], preferred_element_type=jnp.float32)
    kw_ref[...] = k * (we_ref[...] * (1.0 / np.sqrt(_DH)))


def _pre_call(x, Wq, Wk, Wv, We):
    blk = 2000
    grid = _N // blk
    return pl.pallas_call(
        _pre_body,
        grid=(grid,),
        in_specs=[
            pl.BlockSpec((blk, _D), lambda i: (i, 0)),
            pl.BlockSpec((_D, _D), lambda i: (0, 0)),
            pl.BlockSpec((_D, _D), lambda i: (0, 0)),
            pl.BlockSpec((_D, _D), lambda i: (0, 0)),
            pl.BlockSpec((1, _D), lambda i: (0, 0)),
        ],
        out_specs=[
            pl.BlockSpec((blk, _D), lambda i: (i, 0)),
            pl.BlockSpec((blk, _D), lambda i: (i, 0)),
            pl.BlockSpec((blk, _D), lambda i: (i, 0)),
        ],
        out_shape=[
            jax.ShapeDtypeStruct((_N, _D), jnp.float32),
            jax.ShapeDtypeStruct((_N, _D), jnp.float32),
            jax.ShapeDtypeStruct((_N, _D), jnp.float32),
        ],
    )(x, Wq, Wk, Wv, We)


def _sc_body(kw_hbm, v_hbm, q_hbm, src_hbm, dst_hbm, ea_hbm, zeros_hbm,
             owv_hbm, oz_hbm,
             wv_acc, z_acc, kw_v, v_v, q_v, zmsg_v,
             src_v, dst_v, dstz_v, dstm_v, ea_v, sbuf, sem1, sem2, sem3):
    cid = lax.axis_index("c")
    sid = lax.axis_index("s")
    wid = cid * _NSUB + sid

    w0 = sid * _WV_PER_SUB
    z0 = sid * _Z_PER_SUB
    pltpu.sync_copy(zeros_hbm.at[pl.ds(0, _WV_PER_SUB)],
                    wv_acc.at[pl.ds(w0, _WV_PER_SUB)])
    pltpu.sync_copy(zeros_hbm.at[pl.ds(0, _Z_PER_SUB)],
                    z_acc.at[pl.ds(z0, _Z_PER_SUB)])
    pltpu.sync_copy(zeros_hbm.at[pl.ds(0, _CHUNK)], zmsg_v)
    zero16 = jnp.zeros((16,), jnp.float32)
    for r in range(16):
        sbuf[r, pl.ds(0, 16)] = zero16
    plsc.subcore_barrier()

    base = _NCHUNK // _NW
    rem = _NCHUNK % _NW
    start = wid * base + jnp.minimum(wid, rem)
    n_my = base + jnp.where(wid < rem, 1, 0)

    lane = lax.iota(jnp.int32, 16)
    rots = [lax.bitwise_and(lane + dd, 15) for dd in range(16)]

    def chunk_body(i, carry):
        c0 = (start + i) * _CHUNK
        pltpu.sync_copy(src_hbm.at[pl.ds(c0, _CHUNK)], src_v)
        pltpu.sync_copy(dst_hbm.at[pl.ds(c0, _CHUNK)], dst_v)
        pltpu.sync_copy(ea_hbm.at[pl.ds(c0, _CHUNK)], ea_v)
        cp1 = pltpu.async_copy(kw_hbm.at[src_v], kw_v, sem1)
        cp2 = pltpu.async_copy(v_hbm.at[src_v], v_v, sem2)
        cp3 = pltpu.async_copy(q_hbm.at[dst_v], q_v, sem3)

        def split_body(g, carry2):
            dv = dst_v[pl.ds(g * 16, 16)]
            dstz_v[pl.ds(g * 16, 16)] = lax.shift_right_logical(dv, 3)
            dstm_v[pl.ds(g * 16, 16)] = lax.bitwise_and(dv, 7)
            return carry2

        lax.fori_loop(0, _CHUNK // 16, split_body, 0)
        cp1.wait()
        cp2.wait()
        cp3.wait()

        def group_body(g, carry2):
            edge16 = g * 16 + lane
            ea16 = ea_v[pl.ds(g * 16, 16)]
            for h in range(_H):
                prods = []
                for dd in range(16):
                    colv = rots[dd] + (h * 16)
                    a = plsc.load_gather(kw_v, [edge16, colv])
                    b = plsc.load_gather(q_v, [edge16, colv])
                    prods.append(a * b)
                while len(prods) > 1:
                    prods = [prods[k] + prods[k + 1]
                             for k in range(0, len(prods), 2)]
                sh = jnp.exp(jnp.clip(prods[0] * ea16, -5.0, 5.0))
                plsc.store_scatter(sbuf, [lane, lane * 0 + h], sh)

            def edge_body(j, carry3):
                e = g * 16 + j
                svec = sbuf[j, pl.ds(0, 16)]
                m8 = dstm_v[pl.ds(e, 16)][0]
                zmsg_v[e, pl.ds(m8 * 16, 16)] = svec
                for h in range(_H):
                    vh = v_v[e, pl.ds(h * 16, 16)]
                    v_v[e, pl.ds(h * 16, 16)] = vh * svec[h]
                return carry3

            lax.fori_loop(0, 16, edge_body, 0)
            return carry2

        lax.fori_loop(0, _CHUNK // 16, group_body, 0)
        pltpu.sync_copy(v_v, wv_acc.at[dst_v], add=True)
        pltpu.sync_copy(zmsg_v, z_acc.at[dstz_v], add=True)

        def clean_body(j, carry2):
            m8 = dstm_v[pl.ds(j, 16)][0]
            zmsg_v[j, pl.ds(m8 * 16, 16)] = zero16
            return carry2

        lax.fori_loop(0, _CHUNK, clean_body, 0)
        return carry

    lax.fori_loop(0, n_my, chunk_body, 0)
    plsc.subcore_barrier()
    pltpu.sync_copy(wv_acc.at[pl.ds(w0, _WV_PER_SUB)],
                    owv_hbm.at[cid, pl.ds(w0, _WV_PER_SUB)])
    pltpu.sync_copy(z_acc.at[pl.ds(z0, _Z_PER_SUB)],
                    oz_hbm.at[cid, pl.ds(z0, _Z_PER_SUB)])


@functools.partial(
    pl.kernel,
    mesh=plsc.VectorSubcoreMesh(core_axis_name="c", subcore_axis_name="s"),
    compiler_params=pltpu.CompilerParams(needs_layout_passes=False),
    out_type=[
        jax.ShapeDtypeStruct((2, _NPAD, _D), jnp.float32),
        jax.ShapeDtypeStruct((2, _NZ, _D), jnp.float32),
    ],
    scratch_types=[
        pltpu.VMEM_SHARED((_NPAD, _D), jnp.float32),
        pltpu.VMEM_SHARED((_NZ, _D), jnp.float32),
        pltpu.VMEM((_CHUNK, _D), jnp.float32),
        pltpu.VMEM((_CHUNK, _D), jnp.float32),
        pltpu.VMEM((_CHUNK, _D), jnp.float32),
        pltpu.VMEM((_CHUNK, _D), jnp.float32),
        pltpu.VMEM((_CHUNK,), jnp.int32),
        pltpu.VMEM((_CHUNK,), jnp.int32),
        pltpu.VMEM((_CHUNK,), jnp.int32),
        pltpu.VMEM((_CHUNK + 16,), jnp.int32),
        pltpu.VMEM((_CHUNK,), jnp.float32),
        pltpu.VMEM((16, 17), jnp.float32),
        pltpu.SemaphoreType.DMA,
        pltpu.SemaphoreType.DMA,
        pltpu.SemaphoreType.DMA,
    ],
)
def _sc_call(*args):
    _sc_body(*args)


def _post_body(wv_ref, z_ref, x_ref, wo_ref, bo_ref, w1_ref, b1_ref, w2_ref,
               b2_ref, g1_ref, be1_ref, g2_ref, be2_ref, brep_ref, out_ref):
    wv = wv_ref[0] + wv_ref[1]
    z16 = z_ref[0] + z_ref[1]
    zfull = jnp.dot(z16, brep_ref[...], preferred_element_type=jnp.float32)
    h_attn = wv / (zfull + 1e-6)
    h = jnp.dot(h_attn, wo_ref[...], preferred_element_type=jnp.float32)
    h = h + bo_ref[...]
    h = x_ref[...] + h
    m1 = jnp.mean(h, axis=0, keepdims=True)
    v1 = jnp.mean((h - m1) ** 2, axis=0, keepdims=True)
    h = (h - m1) / jnp.sqrt(v1 + 1e-5) * g1_ref[...] + be1_ref[...]
    h2 = jnp.dot(h, w1_ref[...], preferred_element_type=jnp.float32)
    h2 = jnp.maximum(h2 + b1_ref[...], 0.0)
    h2 = jnp.dot(h2, w2_ref[...], preferred_element_type=jnp.float32)
    h2 = h2 + b2_ref[...]
    h = h + h2
    m2 = jnp.mean(h, axis=0, keepdims=True)
    v2 = jnp.mean((h - m2) ** 2, axis=0, keepdims=True)
    out_ref[...] = (h - m2) / jnp.sqrt(v2 + 1e-5) * g2_ref[...] + be2_ref[...]


def _post_call(wv, z, x, Wo, bo, W1, b1, W2, b2, g1, be1, g2, be2, brep):
    return pl.pallas_call(
        _post_body,
        out_shape=jax.ShapeDtypeStruct((_N, _D), jnp.float32),
    )(wv, z, x, Wo, bo, W1, b1, W2, b2, g1, be1, g2, be2, brep)


def kernel(x, edge_index, edge_attr, Wq, Wk, We, Wv, Wo, bo, W1, b1, W2, b2,
           g1, be1, g2, be2):
    q, kw, v = _pre_call(x, Wq, Wk, Wv, We)
    src = edge_index[0]
    dst = edge_index[1]
    ea = edge_attr[:, 0]
    zeros = jnp.zeros((_WV_PER_SUB, _D), jnp.float32)
    owv, oz = _sc_call(kw, v, q, src, dst, ea, zeros)
    wv = owv[:, :_N, :]
    z = oz.reshape(2, _NZ * 8, 16)[:, :_N, :]
    brep = (jnp.arange(_D)[None, :] // _DH == jnp.arange(16)[:, None]).astype(
        jnp.float32)
    return _post_call(wv, z, x, Wo, bo, W1, b1, W2, b2, g1, be1, g2, be2, brep)


# speedup vs baseline: 1.3700x; 1.3700x over previous
import functools

import jax
import jax.numpy as jnp
import numpy as np
from jax import lax
from jax.experimental import pallas as pl
from jax.experimental.pallas import tpu as pltpu
from jax.experimental.pallas import tpu_sc as plsc

_N = 10000
_E = 320000
_D = 128
_H = 8
_DH = 16

_CHUNK = 32
_NCHUNK = _E // _CHUNK
_NW = 32
_NSUB = 16
_NPAD = 10112
_WV_PER_SUB = _NPAD // _NSUB
_NZ = 1280
_Z_PER_SUB = _NZ // _NSUB
_NGRP = _CHUNK // 16


def _pre_body(x_ref, wq_ref, wk_ref, wv_ref, we_ref, q_ref, kw_ref, v_ref):
    x = x_ref[...]
    q_ref[...] = jnp.dot(x, wq_ref[...], preferred_element_type=jnp.float32)
    k = jnp.dot(x, wk_ref[...], preferred_element_type=jnp.float32)
    v_ref[...] = jnp.dot(x, wv_ref[...], preferred_element_type=jnp.float32)
    kw_ref[...] = k * (we_ref[...] * (1.0 / np.sqrt(_DH)))


def _pre_call(x, Wq, Wk, Wv, We):
    blk = 2000
    grid = _N // blk
    return pl.pallas_call(
        _pre_body,
        grid=(grid,),
        in_specs=[
            pl.BlockSpec((blk, _D), lambda i: (i, 0)),
            pl.BlockSpec((_D, _D), lambda i: (0, 0)),
            pl.BlockSpec((_D, _D), lambda i: (0, 0)),
            pl.BlockSpec((_D, _D), lambda i: (0, 0)),
            pl.BlockSpec((1, _D), lambda i: (0, 0)),
        ],
        out_specs=[
            pl.BlockSpec((blk, _D), lambda i: (i, 0)),
            pl.BlockSpec((blk, _D), lambda i: (i, 0)),
            pl.BlockSpec((blk, _D), lambda i: (i, 0)),
        ],
        out_shape=[
            jax.ShapeDtypeStruct((_N, _D), jnp.float32),
            jax.ShapeDtypeStruct((_N, _D), jnp.float32),
            jax.ShapeDtypeStruct((_N, _D), jnp.float32),
        ],
    )(x, Wq, Wk, Wv, We)


def _sc_body(kw_hbm, v_hbm, q_hbm, src_hbm, dst_hbm, ea_hbm, zeros_hbm,
             owv_hbm, oz_hbm,
             wv_acc, z_acc, kw_v, v_v, q_v, zmsg_v,
             src_v, dst_v, dstadd_v, dstz_v, dstzadd_v, dstm_v, ea_v, sbuf,
             gsem0, gsem1, isem0, isem1, asem0, asem1):
    cid = lax.axis_index("c")
    sid = lax.axis_index("s")
    wid = cid * _NSUB + sid

    w0 = sid * _WV_PER_SUB
    z0 = sid * _Z_PER_SUB
    pltpu.sync_copy(zeros_hbm.at[pl.ds(0, _WV_PER_SUB)],
                    wv_acc.at[pl.ds(w0, _WV_PER_SUB)])
    pltpu.sync_copy(zeros_hbm.at[pl.ds(0, _Z_PER_SUB)],
                    z_acc.at[pl.ds(z0, _Z_PER_SUB)])
    pltpu.sync_copy(zeros_hbm.at[pl.ds(0, _CHUNK)], zmsg_v.at[0])
    pltpu.sync_copy(zeros_hbm.at[pl.ds(0, _CHUNK)], zmsg_v.at[1])
    zero16 = jnp.zeros((16,), jnp.float32)
    for r in range(16):
        sbuf[r, pl.ds(0, 16)] = zero16
    plsc.subcore_barrier()

    base = _NCHUNK // _NW
    rem = _NCHUNK % _NW
    first = wid * base + jnp.minimum(wid, rem)
    n_my = base + jnp.where(wid < rem, 1, 0)

    lane = lax.iota(jnp.int32, 16)
    rots = [lax.bitwise_and(lane + dd, 15) for dd in range(16)]
    gsems = (gsem0, gsem1)
    isems = (isem0, isem1)
    asems = (asem0, asem1)

    def issue_idx(c, ph):
        c0 = (first + c) * _CHUNK
        pltpu.async_copy(src_hbm.at[pl.ds(c0, _CHUNK)], src_v.at[ph],
                         isems[ph])
        pltpu.async_copy(dst_hbm.at[pl.ds(c0, _CHUNK)], dst_v.at[ph],
                         isems[ph])
        pltpu.async_copy(ea_hbm.at[pl.ds(c0, _CHUNK)], ea_v.at[ph],
                         isems[ph])

    def issue_gathers(ph):
        pltpu.async_copy(kw_hbm.at[src_v.at[ph]], kw_v.at[ph], gsems[ph])
        pltpu.async_copy(v_hbm.at[src_v.at[ph]], v_v.at[ph], gsems[ph])
        pltpu.async_copy(q_hbm.at[dst_v.at[ph]], q_v.at[ph], gsems[ph])

    def drain_idx(ph):
        pltpu.make_async_copy(src_hbm.at[pl.ds(0, _CHUNK)], src_v.at[ph],
                              isems[ph]).wait()
        pltpu.make_async_copy(dst_hbm.at[pl.ds(0, _CHUNK)], dst_v.at[ph],
                              isems[ph]).wait()
        pltpu.make_async_copy(ea_hbm.at[pl.ds(0, _CHUNK)], ea_v.at[ph],
                              isems[ph]).wait()

    def drain_gathers(ph):
        pltpu.make_async_copy(kw_hbm.at[pl.ds(0, _CHUNK)], kw_v.at[ph],
                              gsems[ph]).wait()
        pltpu.make_async_copy(v_hbm.at[pl.ds(0, _CHUNK)], v_v.at[ph],
                              gsems[ph]).wait()
        pltpu.make_async_copy(q_hbm.at[pl.ds(0, _CHUNK)], q_v.at[ph],
                              gsems[ph]).wait()

    def drain_adds(ph):
        pltpu.make_async_copy(v_v.at[ph], wv_acc.at[pl.ds(0, _CHUNK)],
                              asems[ph]).wait()
        pltpu.make_async_copy(zmsg_v.at[ph], z_acc.at[pl.ds(0, _CHUNK)],
                              asems[ph]).wait()

    def compute_chunk(ph):
        def split_body(g, carry2):
            dv = dst_v[ph, pl.ds(g * 16, 16)]
            dstz_v[ph, pl.ds(g * 16, 16)] = lax.shift_right_logical(dv, 3)
            dstm_v[ph, pl.ds(g * 16, 16)] = lax.bitwise_and(dv, 7)
            dstadd_v[ph, pl.ds(g * 16, 16)] = dv
            return carry2

        lax.fori_loop(0, _NGRP, split_body, 0)

        def group_body(g, carry2):
            edge16 = g * 16 + lane
            ea16 = ea_v[ph, pl.ds(g * 16, 16)]
            def head_body(h, carry3):
                h16 = h * 16
                prods = []
                for dd in range(16):
                    colv = rots[dd] + h16
                    a = plsc.load_gather(kw_v.at[ph], [edge16, colv])
                    b = plsc.load_gather(q_v.at[ph], [edge16, colv])
                    prods.append(a * b)
                while len(prods) > 1:
                    prods = [prods[k] + prods[k + 1]
                             for k in range(0, len(prods), 2)]
                sh = jnp.exp(jnp.clip(prods[0] * ea16, -5.0, 5.0))
                plsc.store_scatter(sbuf, [lane, lane * 0 + h], sh)
                return carry3

            lax.fori_loop(0, _H, head_body, 0)

            def edge_body(j, carry3):
                e = g * 16 + j
                svec = sbuf[j, pl.ds(0, 16)]
                m8 = dstm_v[ph, pl.ds(e, 16)][0]
                zmsg_v[ph, e, pl.ds(m8 * 16, 16)] = svec
                for h in range(_H):
                    vh = v_v[ph, e, pl.ds(h * 16, 16)]
                    v_v[ph, e, pl.ds(h * 16, 16)] = vh * svec[h]
                return carry3

            lax.fori_loop(0, 16, edge_body, 0)
            return carry2

        lax.fori_loop(0, _NGRP, group_body, 0)
        dstzadd_v[ph, pl.ds(0, 16)] = dstz_v[ph, pl.ds(0, 16)]
        dstzadd_v[ph, pl.ds(16, 16)] = dstz_v[ph, pl.ds(16, 16)]
        pltpu.async_copy(v_v.at[ph], wv_acc.at[dstadd_v.at[ph]], asems[ph],
                         add=True)
        pltpu.async_copy(zmsg_v.at[ph], z_acc.at[dstzadd_v.at[ph]],
                         asems[ph], add=True)

    def clean_zmsg(ph):
        def clean_body(j, carry2):
            m8 = dstm_v[ph, pl.ds(j, 16)][0]
            zmsg_v[ph, j, pl.ds(m8 * 16, 16)] = zero16
            return carry2

        lax.fori_loop(0, _CHUNK, clean_body, 0)

    def step(i, ph):
        qh = 1 - ph
        @pl.when(jnp.logical_and(i >= 1, i <= n_my))
        def _():
            drain_adds(qh)
            clean_zmsg(qh)

        @pl.when(i + 1 <= n_my - 1)
        def _():
            drain_idx(qh)
            issue_gathers(qh)

        @pl.when(i <= n_my - 1)
        def _():
            drain_gathers(ph)
            compute_chunk(ph)

        @pl.when(i + 2 <= n_my - 1)
        def _():
            issue_idx(i + 2, ph)

    issue_idx(0, 0)
    drain_idx(0)
    issue_gathers(0)
    issue_idx(1, 1)

    def pair_body(t, carry):
        step(2 * t, 0)
        step(2 * t + 1, 1)
        return carry

    max_chunks = base + 1
    lax.fori_loop(0, (max_chunks + 2) // 2, pair_body, 0)
    plsc.subcore_barrier()
    pltpu.sync_copy(wv_acc.at[pl.ds(w0, _WV_PER_SUB)],
                    owv_hbm.at[cid, pl.ds(w0, _WV_PER_SUB)])
    pltpu.sync_copy(z_acc.at[pl.ds(z0, _Z_PER_SUB)],
                    oz_hbm.at[cid, pl.ds(z0, _Z_PER_SUB)])


@functools.partial(
    pl.kernel,
    mesh=plsc.VectorSubcoreMesh(core_axis_name="c", subcore_axis_name="s"),
    compiler_params=pltpu.CompilerParams(needs_layout_passes=False),
    out_type=[
        jax.ShapeDtypeStruct((2, _NPAD, _D), jnp.float32),
        jax.ShapeDtypeStruct((2, _NZ, _D), jnp.float32),
    ],
    scratch_types=[
        pltpu.VMEM_SHARED((_NPAD, _D), jnp.float32),
        pltpu.VMEM_SHARED((_NZ, _D), jnp.float32),
        pltpu.VMEM((2, _CHUNK, _D), jnp.float32),
        pltpu.VMEM((2, _CHUNK, _D), jnp.float32),
        pltpu.VMEM((2, _CHUNK, _D), jnp.float32),
        pltpu.VMEM((2, _CHUNK, _D), jnp.float32),
        pltpu.VMEM((2, _CHUNK), jnp.int32),
        pltpu.VMEM((2, _CHUNK), jnp.int32),
        pltpu.VMEM((2, _CHUNK), jnp.int32),
        pltpu.VMEM((2, _CHUNK), jnp.int32),
        pltpu.VMEM((2, _CHUNK), jnp.int32),
        pltpu.VMEM((2, _CHUNK + 16), jnp.int32),
        pltpu.VMEM((2, _CHUNK), jnp.float32),
        pltpu.VMEM((16, 17), jnp.float32),
        pltpu.SemaphoreType.DMA,
        pltpu.SemaphoreType.DMA,
        pltpu.SemaphoreType.DMA,
        pltpu.SemaphoreType.DMA,
        pltpu.SemaphoreType.DMA,
        pltpu.SemaphoreType.DMA,
    ],
)
def _sc_call(*args):
    _sc_body(*args)


def _post_body(wv_ref, z_ref, x_ref, wo_ref, bo_ref, w1_ref, b1_ref, w2_ref,
               b2_ref, g1_ref, be1_ref, g2_ref, be2_ref, brep_ref, out_ref):
    wv = wv_ref[0] + wv_ref[1]
    z16 = z_ref[0] + z_ref[1]
    zfull = jnp.dot(z16, brep_ref[...], preferred_element_type=jnp.float32)
    h_attn = wv / (zfull + 1e-6)
    h = jnp.dot(h_attn, wo_ref[...], preferred_element_type=jnp.float32)
    h = h + bo_ref[...]
    h = x_ref[...] + h
    m1 = jnp.mean(h, axis=0, keepdims=True)
    v1 = jnp.mean((h - m1) ** 2, axis=0, keepdims=True)
    h = (h - m1) / jnp.sqrt(v1 + 1e-5) * g1_ref[...] + be1_ref[...]
    h2 = jnp.dot(h, w1_ref[...], preferred_element_type=jnp.float32)
    h2 = jnp.maximum(h2 + b1_ref[...], 0.0)
    h2 = jnp.dot(h2, w2_ref[...], preferred_element_type=jnp.float32)
    h2 = h2 + b2_ref[...]
    h = h + h2
    m2 = jnp.mean(h, axis=0, keepdims=True)
    v2 = jnp.mean((h - m2) ** 2, axis=0, keepdims=True)
    out_ref[...] = (h - m2) / jnp.sqrt(v2 + 1e-5) * g2_ref[...] + be2_ref[...]


def _post_call(wv, z, x, Wo, bo, W1, b1, W2, b2, g1, be1, g2, be2, brep):
    return pl.pallas_call(
        _post_body,
        out_shape=jax.ShapeDtypeStruct((_N, _D), jnp.float32),
    )(wv, z, x, Wo, bo, W1, b1, W2, b2, g1, be1, g2, be2, brep)


def kernel(x, edge_index, edge_attr, Wq, Wk, We, Wv, Wo, bo, W1, b1, W2, b2,
           g1, be1, g2, be2):
    q, kw, v = _pre_call(x, Wq, Wk, Wv, We)
    src = edge_index[0]
    dst = edge_index[1]
    ea = edge_attr[:, 0]
    zeros = jnp.zeros((_WV_PER_SUB, _D), jnp.float32)
    owv, oz = _sc_call(kw, v, q, src, dst, ea, zeros)
    wv = owv[:, :_N, :]
    z = oz.reshape(2, _NZ * 8, 16)[:, :_N, :]
    brep = (jnp.arange(_D)[None, :] // _DH == jnp.arange(16)[:, None]).astype(
        jnp.float32)
    return _post_call(wv, z, x, Wo, bo, W1, b1, W2, b2, g1, be1, g2, be2, brep)


# speedup vs baseline: 1.3759x; 1.0043x over previous
import functools

import jax
import jax.numpy as jnp
import numpy as np
from jax import lax
from jax.experimental import pallas as pl
from jax.experimental.pallas import tpu as pltpu
from jax.experimental.pallas import tpu_sc as plsc

_N = 10000
_E = 320000
_D = 128
_H = 8
_DH = 16

_CHUNK = 32
_NCHUNK = _E // _CHUNK
_NW = 32
_NSUB = 16
_NPAD = 10112
_WV_PER_SUB = _NPAD // _NSUB
_NZ = 1280
_Z_PER_SUB = _NZ // _NSUB
_NGRP = _CHUNK // 16


def _pre_body(x_ref, wq_ref, wk_ref, wv_ref, we_ref, q_ref, kw_ref, v_ref):
    x = x_ref[...]
    q_ref[...] = jnp.dot(x, wq_ref[...], preferred_element_type=jnp.float32)
    k = jnp.dot(x, wk_ref[...], preferred_element_type=jnp.float32)
    v_ref[...] = jnp.dot(x, wv_ref[...], preferred_element_type=jnp.float32)
    kw_ref[...] = k * (we_ref[...] * (1.0 / np.sqrt(_DH)))


def _pre_call(x, Wq, Wk, Wv, We):
    blk = 2000
    grid = _N // blk
    return pl.pallas_call(
        _pre_body,
        grid=(grid,),
        in_specs=[
            pl.BlockSpec((blk, _D), lambda i: (i, 0)),
            pl.BlockSpec((_D, _D), lambda i: (0, 0)),
            pl.BlockSpec((_D, _D), lambda i: (0, 0)),
            pl.BlockSpec((_D, _D), lambda i: (0, 0)),
            pl.BlockSpec((1, _D), lambda i: (0, 0)),
        ],
        out_specs=[
            pl.BlockSpec((blk, _D), lambda i: (i, 0)),
            pl.BlockSpec((blk, _D), lambda i: (i, 0)),
            pl.BlockSpec((blk, _D), lambda i: (i, 0)),
        ],
        out_shape=[
            jax.ShapeDtypeStruct((_N, _D), jnp.float32),
            jax.ShapeDtypeStruct((_N, _D), jnp.float32),
            jax.ShapeDtypeStruct((_N, _D), jnp.float32),
        ],
    )(x, Wq, Wk, Wv, We)


def _sc_body(kw_hbm, v_hbm, q_hbm, src_hbm, dst_hbm, ea_hbm, zeros_hbm,
             owv_hbm, oz_hbm,
             wv_acc, z_acc, kw_v, v_v, q_v, zmsg_v,
             src_v, dst_v, dstadd_v, dstz_v, dstzadd_v, dstm_v, ea_v, sbuf,
             gsem0, gsem1, isem0, isem1, asem0, asem1):
    cid = lax.axis_index("c")
    sid = lax.axis_index("s")
    wid = cid * _NSUB + sid

    w0 = sid * _WV_PER_SUB
    z0 = sid * _Z_PER_SUB
    pltpu.sync_copy(zeros_hbm.at[pl.ds(0, _WV_PER_SUB)],
                    wv_acc.at[pl.ds(w0, _WV_PER_SUB)])
    pltpu.sync_copy(zeros_hbm.at[pl.ds(0, _Z_PER_SUB)],
                    z_acc.at[pl.ds(z0, _Z_PER_SUB)])
    pltpu.sync_copy(zeros_hbm.at[pl.ds(0, _CHUNK)], zmsg_v.at[0])
    pltpu.sync_copy(zeros_hbm.at[pl.ds(0, _CHUNK)], zmsg_v.at[1])
    zero16 = jnp.zeros((16,), jnp.float32)
    for r in range(16):
        sbuf[r, pl.ds(0, 16)] = zero16
    plsc.subcore_barrier()

    base = _NCHUNK // _NW
    rem = _NCHUNK % _NW
    first = wid * base + jnp.minimum(wid, rem)
    n_my = base + jnp.where(wid < rem, 1, 0)

    lane = lax.iota(jnp.int32, 16)
    rots = [lax.bitwise_and(lane + dd, 15) for dd in range(16)]
    gsems = (gsem0, gsem1)
    isems = (isem0, isem1)
    asems = (asem0, asem1)

    def issue_idx(c, ph):
        c0 = (first + c) * _CHUNK
        pltpu.async_copy(src_hbm.at[pl.ds(c0, _CHUNK)], src_v.at[ph],
                         isems[ph])
        pltpu.async_copy(dst_hbm.at[pl.ds(c0, _CHUNK)], dst_v.at[ph],
                         isems[ph])
        pltpu.async_copy(ea_hbm.at[pl.ds(c0, _CHUNK)], ea_v.at[ph],
                         isems[ph])

    def issue_gathers(ph):
        pltpu.async_copy(kw_hbm.at[src_v.at[ph]], kw_v.at[ph], gsems[ph])
        pltpu.async_copy(v_hbm.at[src_v.at[ph]], v_v.at[ph], gsems[ph])
        pltpu.async_copy(q_hbm.at[dst_v.at[ph]], q_v.at[ph], gsems[ph])

    def drain_idx(ph):
        pltpu.make_async_copy(src_hbm.at[pl.ds(0, _CHUNK)], src_v.at[ph],
                              isems[ph]).wait()
        pltpu.make_async_copy(dst_hbm.at[pl.ds(0, _CHUNK)], dst_v.at[ph],
                              isems[ph]).wait()
        pltpu.make_async_copy(ea_hbm.at[pl.ds(0, _CHUNK)], ea_v.at[ph],
                              isems[ph]).wait()

    def drain_gathers(ph):
        pltpu.make_async_copy(kw_hbm.at[pl.ds(0, _CHUNK)], kw_v.at[ph],
                              gsems[ph]).wait()
        pltpu.make_async_copy(v_hbm.at[pl.ds(0, _CHUNK)], v_v.at[ph],
                              gsems[ph]).wait()
        pltpu.make_async_copy(q_hbm.at[pl.ds(0, _CHUNK)], q_v.at[ph],
                              gsems[ph]).wait()

    def drain_adds(ph):
        pltpu.make_async_copy(v_v.at[ph], wv_acc.at[pl.ds(0, _CHUNK)],
                              asems[ph]).wait()
        pltpu.make_async_copy(zmsg_v.at[ph], z_acc.at[pl.ds(0, _CHUNK)],
                              asems[ph]).wait()

    def compute_chunk(ph):
        def split_body(g, carry2):
            dv = dst_v[ph, pl.ds(g * 16, 16)]
            dstz_v[ph, pl.ds(g * 16, 16)] = lax.shift_right_logical(dv, 3)
            dstm_v[ph, pl.ds(g * 16, 16)] = lax.bitwise_and(dv, 7)
            dstadd_v[ph, pl.ds(g * 16, 16)] = dv
            return carry2

        lax.fori_loop(0, _NGRP, split_body, 0)

        def group_body(g, carry2):
            edge16 = g * 16 + lane
            ea16 = ea_v[ph, pl.ds(g * 16, 16)]
            def head_body(h, carry3):
                h16 = h * 16
                prods = []
                for dd in range(16):
                    colv = rots[dd] + h16
                    a = plsc.load_gather(kw_v.at[ph], [edge16, colv])
                    b = plsc.load_gather(q_v.at[ph], [edge16, colv])
                    prods.append(a * b)
                while len(prods) > 1:
                    prods = [prods[k] + prods[k + 1]
                             for k in range(0, len(prods), 2)]
                sh = jnp.exp(jnp.clip(prods[0] * ea16, -5.0, 5.0))
                plsc.store_scatter(sbuf, [lane, lane * 0 + h], sh)
                for dd in range(16):
                    colv = rots[dd] + h16
                    vv = plsc.load_gather(v_v.at[ph], [edge16, colv])
                    plsc.store_scatter(v_v.at[ph], [edge16, colv], vv * sh)
                return carry3

            lax.fori_loop(0, _H, head_body, 0)

            def edge_body(j, carry3):
                e = g * 16 + j
                svec = sbuf[j, pl.ds(0, 16)]
                m8 = dstm_v[ph, pl.ds(e, 16)][0]
                zmsg_v[ph, e, pl.ds(m8 * 16, 16)] = svec
                return carry3

            lax.fori_loop(0, 16, edge_body, 0)
            return carry2

        lax.fori_loop(0, _NGRP, group_body, 0)
        dstzadd_v[ph, pl.ds(0, 16)] = dstz_v[ph, pl.ds(0, 16)]
        dstzadd_v[ph, pl.ds(16, 16)] = dstz_v[ph, pl.ds(16, 16)]
        pltpu.async_copy(v_v.at[ph], wv_acc.at[dstadd_v.at[ph]], asems[ph],
                         add=True)
        pltpu.async_copy(zmsg_v.at[ph], z_acc.at[dstzadd_v.at[ph]],
                         asems[ph], add=True)

    def clean_zmsg(ph):
        def clean_body(j, carry2):
            m8 = dstm_v[ph, pl.ds(j, 16)][0]
            zmsg_v[ph, j, pl.ds(m8 * 16, 16)] = zero16
            return carry2

        lax.fori_loop(0, _CHUNK, clean_body, 0)

    def step(i, ph):
        qh = 1 - ph
        @pl.when(jnp.logical_and(i >= 1, i <= n_my))
        def _():
            drain_adds(qh)
            clean_zmsg(qh)

        @pl.when(i + 1 <= n_my - 1)
        def _():
            drain_idx(qh)
            issue_gathers(qh)

        @pl.when(i <= n_my - 1)
        def _():
            drain_gathers(ph)
            compute_chunk(ph)

        @pl.when(i + 2 <= n_my - 1)
        def _():
            issue_idx(i + 2, ph)

    issue_idx(0, 0)
    drain_idx(0)
    issue_gathers(0)
    issue_idx(1, 1)

    def pair_body(t, carry):
        step(2 * t, 0)
        step(2 * t + 1, 1)
        return carry

    max_chunks = base + 1
    lax.fori_loop(0, (max_chunks + 2) // 2, pair_body, 0)
    plsc.subcore_barrier()
    pltpu.sync_copy(wv_acc.at[pl.ds(w0, _WV_PER_SUB)],
                    owv_hbm.at[cid, pl.ds(w0, _WV_PER_SUB)])
    pltpu.sync_copy(z_acc.at[pl.ds(z0, _Z_PER_SUB)],
                    oz_hbm.at[cid, pl.ds(z0, _Z_PER_SUB)])


@functools.partial(
    pl.kernel,
    mesh=plsc.VectorSubcoreMesh(core_axis_name="c", subcore_axis_name="s"),
    compiler_params=pltpu.CompilerParams(needs_layout_passes=False),
    out_type=[
        jax.ShapeDtypeStruct((2, _NPAD, _D), jnp.float32),
        jax.ShapeDtypeStruct((2, _NZ, _D), jnp.float32),
    ],
    scratch_types=[
        pltpu.VMEM_SHARED((_NPAD, _D), jnp.float32),
        pltpu.VMEM_SHARED((_NZ, _D), jnp.float32),
        pltpu.VMEM((2, _CHUNK, _D), jnp.float32),
        pltpu.VMEM((2, _CHUNK, _D), jnp.float32),
        pltpu.VMEM((2, _CHUNK, _D), jnp.float32),
        pltpu.VMEM((2, _CHUNK, _D), jnp.float32),
        pltpu.VMEM((2, _CHUNK), jnp.int32),
        pltpu.VMEM((2, _CHUNK), jnp.int32),
        pltpu.VMEM((2, _CHUNK), jnp.int32),
        pltpu.VMEM((2, _CHUNK), jnp.int32),
        pltpu.VMEM((2, _CHUNK), jnp.int32),
        pltpu.VMEM((2, _CHUNK + 16), jnp.int32),
        pltpu.VMEM((2, _CHUNK), jnp.float32),
        pltpu.VMEM((16, 17), jnp.float32),
        pltpu.SemaphoreType.DMA,
        pltpu.SemaphoreType.DMA,
        pltpu.SemaphoreType.DMA,
        pltpu.SemaphoreType.DMA,
        pltpu.SemaphoreType.DMA,
        pltpu.SemaphoreType.DMA,
    ],
)
def _sc_call(*args):
    _sc_body(*args)


def _post_body(wv_ref, z_ref, x_ref, wo_ref, bo_ref, w1_ref, b1_ref, w2_ref,
               b2_ref, g1_ref, be1_ref, g2_ref, be2_ref, brep_ref, out_ref):
    wv = wv_ref[0] + wv_ref[1]
    z16 = z_ref[0] + z_ref[1]
    zfull = jnp.dot(z16, brep_ref[...], preferred_element_type=jnp.float32)
    h_attn = wv / (zfull + 1e-6)
    h = jnp.dot(h_attn, wo_ref[...], preferred_element_type=jnp.float32)
    h = h + bo_ref[...]
    h = x_ref[...] + h
    m1 = jnp.mean(h, axis=0, keepdims=True)
    v1 = jnp.mean((h - m1) ** 2, axis=0, keepdims=True)
    h = (h - m1) / jnp.sqrt(v1 + 1e-5) * g1_ref[...] + be1_ref[...]
    h2 = jnp.dot(h, w1_ref[...], preferred_element_type=jnp.float32)
    h2 = jnp.maximum(h2 + b1_ref[...], 0.0)
    h2 = jnp.dot(h2, w2_ref[...], preferred_element_type=jnp.float32)
    h2 = h2 + b2_ref[...]
    h = h + h2
    m2 = jnp.mean(h, axis=0, keepdims=True)
    v2 = jnp.mean((h - m2) ** 2, axis=0, keepdims=True)
    out_ref[...] = (h - m2) / jnp.sqrt(v2 + 1e-5) * g2_ref[...] + be2_ref[...]


def _post_call(wv, z, x, Wo, bo, W1, b1, W2, b2, g1, be1, g2, be2, brep):
    return pl.pallas_call(
        _post_body,
        out_shape=jax.ShapeDtypeStruct((_N, _D), jnp.float32),
    )(wv, z, x, Wo, bo, W1, b1, W2, b2, g1, be1, g2, be2, brep)


def kernel(x, edge_index, edge_attr, Wq, Wk, We, Wv, Wo, bo, W1, b1, W2, b2,
           g1, be1, g2, be2):
    q, kw, v = _pre_call(x, Wq, Wk, Wv, We)
    src = edge_index[0]
    dst = edge_index[1]
    ea = edge_attr[:, 0]
    zeros = jnp.zeros((_WV_PER_SUB, _D), jnp.float32)
    owv, oz = _sc_call(kw, v, q, src, dst, ea, zeros)
    wv = owv[:, :_N, :]
    z = oz.reshape(2, _NZ * 8, 16)[:, :_N, :]
    brep = (jnp.arange(_D)[None, :] // _DH == jnp.arange(16)[:, None]).astype(
        jnp.float32)
    return _post_call(wv, z, x, Wo, bo, W1, b1, W2, b2, g1, be1, g2, be2, brep)
